# baseline (speedup 1.0000x reference)
import jax
import jax.numpy as jnp
from jax import lax
from jax.experimental import pallas as pl
from jax.experimental.pallas import tpu as pltpu
from jax.experimental.pallas import tpu_sc as plsc

B = 16
P = 8
H = 128
N = 10000
E = 160000
NPG = N // B

EBLK = 2000
NEBLK = E // EBLK

NW = 32
CHUNK = 125
NCH = 40
EPW = NCH * CHUNK
GPAD = 16


def _dot3(a, b):
    def hi(x):
        xi = lax.bitcast_convert_type(x, jnp.int32)
        return lax.bitcast_convert_type(
            jnp.bitwise_and(xi, jnp.int32(-65536)), jnp.float32)

    ahi = hi(a)
    alo = hi(a - ahi)
    bhi = hi(b)
    blo = hi(b - bhi)

    def d(x, y):
        return lax.dot_general(x, y, (((1,), (1,)), ((), ())),
                               preferred_element_type=jnp.float32,
                               precision=lax.Precision.HIGHEST)

    return d(ahi, bhi) + d(ahi, blo) + d(alo, bhi)


def _elu(x):
    return jnp.where(x > 0, x, jnp.exp(jnp.minimum(x, 0.0)) - 1.0)


def _sc_gather_body(dist_hbm, src_hbm, out_hbm, dist_v, src_v, dsrc_v):
    c = lax.axis_index("c")
    s = lax.axis_index("s")
    wid = s * 2 + c
    base = wid * EPW
    pltpu.sync_copy(dist_hbm, dist_v)
    pltpu.sync_copy(src_hbm.at[pl.ds(base, EPW + GPAD)], src_v)

    def gather(i, carry):
        idx = src_v[pl.ds(i * 16, 16)]
        dsrc_v[pl.ds(i * 16, 16)] = plsc.load_gather(dist_v, [idx])
        return carry

    lax.fori_loop(0, (EPW + 15) // 16, gather, 0)
    pltpu.sync_copy(dsrc_v.at[pl.ds(0, EPW)], out_hbm.at[pl.ds(base, EPW)])


def _sc_gather_call(distribution, src_pad):
    mesh = plsc.VectorSubcoreMesh(core_axis_name="c", subcore_axis_name="s")
    f = pl.kernel(
        _sc_gather_body,
        out_type=jax.ShapeDtypeStruct((E,), jnp.float32),
        mesh=mesh,
        scratch_types=[
            pltpu.VMEM((N,), jnp.float32),
            pltpu.VMEM((EPW + GPAD,), jnp.int32),
            pltpu.VMEM((EPW + GPAD,), jnp.float32),
        ],
        compiler_params=pltpu.CompilerParams(needs_layout_passes=False),
    )
    return f(distribution, src_pad)


def _edge_body(eg_ref, dsrc_ref, ea_ref, instr_ref, w7_ref, out_ref):
    oh = (eg_ref[0] == lax.broadcasted_iota(jnp.int32, (1, B), 1)).astype(jnp.float32)
    instr_g = jnp.dot(oh, instr_ref[...], preferred_element_type=jnp.float32)
    es = lax.dot_general(ea_ref[0], w7_ref[...], (((1,), (0,)), ((), ())),
                         preferred_element_type=jnp.float32)
    out_ref[0] = dsrc_ref[0] * _elu(instr_g * es)


def _edge_call(eg3, dsrc3, ea3, instruction, w7):
    return pl.pallas_call(
        _edge_body,
        grid=(NEBLK,),
        in_specs=[
            pl.BlockSpec((1, EBLK, 1), lambda i: (i, 0, 0)),
            pl.BlockSpec((1, EBLK, 1), lambda i: (i, 0, 0)),
            pl.BlockSpec((1, EBLK, H), lambda i: (i, 0, 0)),
            pl.BlockSpec((B, H), lambda i: (0, 0)),
            pl.BlockSpec((H, H), lambda i: (0, 0)),
        ],
        out_specs=pl.BlockSpec((1, EBLK, H), lambda i: (i, 0, 0)),
        out_shape=jax.ShapeDtypeStruct((NEBLK, EBLK, H), jnp.float32),
    )(eg3, dsrc3, ea3, instruction, w7)


def _sc_scatter_body(msgs_hbm, dst_hbm, zeros_hbm, out_hbm, dst_v, buf_v, acc):
    c = lax.axis_index("c")
    s = lax.axis_index("s")
    wid = s * 2 + c
    pltpu.sync_copy(dst_hbm.at[pl.ds(wid * NCH, NCH)], dst_v)
    pltpu.sync_copy(zeros_hbm, acc.at[pl.ds(s * NPG, NPG)])
    plsc.subcore_barrier()

    def scatter(j, carry):
        row0 = wid * EPW + j * CHUNK
        pltpu.sync_copy(msgs_hbm.at[pl.ds(row0, CHUNK)], buf_v)
        pltpu.sync_copy(buf_v, acc.at[dst_v.at[j]], add=True)
        return carry

    lax.fori_loop(0, NCH, scatter, 0)
    plsc.subcore_barrier()
    pltpu.sync_copy(acc.at[pl.ds(s * NPG, NPG)],
                    out_hbm.at[c, pl.ds(s * NPG, NPG)])


def _sc_scatter_call(msgs, dst2, zeros_np):
    mesh = plsc.VectorSubcoreMesh(core_axis_name="c", subcore_axis_name="s")
    f = pl.kernel(
        _sc_scatter_body,
        out_type=jax.ShapeDtypeStruct((2, N, H), jnp.float32),
        mesh=mesh,
        scratch_types=[
            pltpu.VMEM((NCH, CHUNK), jnp.int32),
            pltpu.VMEM((CHUNK, H), jnp.float32),
            pltpu.VMEM_SHARED((N, H), jnp.float32),
        ],
        compiler_params=pltpu.CompilerParams(needs_layout_passes=False,
                                             use_tc_tiling_on_sc=False),
    )
    return f(msgs, dst2, zeros_np)


def _node_body(na_ref, instr_ref, pe_ref, wsp_ref, ns_ref, ps_ref):
    g = pl.program_id(0)
    instr_row = instr_ref[pl.ds(g, 1), :]
    logits = lax.dot_general(instr_row, pe_ref[...], (((1,), (1,)), ((), ())),
                             preferred_element_type=jnp.float32)
    m = jnp.max(logits, axis=1, keepdims=True)
    ex = jnp.exp(logits - m)
    ps = ex / jnp.sum(ex, axis=1, keepdims=True)
    ps_ref[0] = ps
    z = jnp.zeros((NPG, H), dtype=jnp.float32)
    for p in range(P - 1):
        wx = lax.dot_general(na_ref[0, :, p, :], wsp_ref[p],
                             (((1,), (0,)), ((), ())),
                             preferred_element_type=jnp.float32)
        z = z + ps[:, p:p + 1] * instr_row * wx
    ns_ref[0] = _elu(z)


def _node_call(na4, instruction, prop_embeds, Ws_property):
    return pl.pallas_call(
        _node_body,
        grid=(B,),
        in_specs=[
            pl.BlockSpec((1, NPG, P - 1, H), lambda g: (g, 0, 0, 0)),
            pl.BlockSpec((B, H), lambda g: (0, 0)),
            pl.BlockSpec((P, H), lambda g: (0, 0)),
            pl.BlockSpec((P - 1, H, H), lambda g: (0, 0, 0)),
        ],
        out_specs=[
            pl.BlockSpec((1, NPG, H), lambda g: (g, 0, 0)),
            pl.BlockSpec((1, 1, P), lambda g: (g, 0, 0)),
        ],
        out_shape=[
            jax.ShapeDtypeStruct((B, NPG, H), jnp.float32),
            jax.ShapeDtypeStruct((B, 1, P), jnp.float32),
        ],
    )(na4, instruction, prop_embeds, Ws_property)


MVB = 1000
NMVB = N // MVB


def _matvec_body(ns_ref, aggp_ref, wst_ref, wrel_ref, s_ref, r_ref):
    s_full = lax.dot_general(ns_ref[0], wst_ref[...], (((1,), (0,)), ((), ())),
                             preferred_element_type=jnp.float32)
    s_ref[0] = s_full[:, 0:1]
    agg = aggp_ref[0, 0] + aggp_ref[1, 0]
    r_full = lax.dot_general(agg, wrel_ref[...], (((1,), (0,)), ((), ())),
                             preferred_element_type=jnp.float32)
    r_ref[0] = r_full[:, 0:1]


def _matvec_call(ns3, aggp4, wst, wrel):
    return pl.pallas_call(
        _matvec_body,
        grid=(NMVB,),
        in_specs=[
            pl.BlockSpec((1, MVB, H), lambda i: (i, 0, 0)),
            pl.BlockSpec((2, 1, MVB, H), lambda i: (0, i, 0, 0)),
            pl.BlockSpec((H, H), lambda i: (0, 0)),
            pl.BlockSpec((H, H), lambda i: (0, 0)),
        ],
        out_specs=[
            pl.BlockSpec((1, MVB, 1), lambda i: (i, 0, 0)),
            pl.BlockSpec((1, MVB, 1), lambda i: (i, 0, 0)),
        ],
        out_shape=[
            jax.ShapeDtypeStruct((NMVB, MVB, 1), jnp.float32),
            jax.ShapeDtypeStruct((NMVB, MVB, 1), jnp.float32),
        ],
    )(ns3, aggp4, wst, wrel)


def _finish_body(s_ref, r_ref, ps_ref, out_ref):
    def colsoft(x):
        ex = jnp.exp(x - jnp.max(x))
        return ex / jnp.sum(ex)

    gate = ps_ref[0][0:1, P - 1:P]
    out_ref[0] = gate * colsoft(r_ref[0]) + (1.0 - gate) * colsoft(s_ref[0])


def _finish_call(s3, r3, ps3):
    return pl.pallas_call(
        _finish_body,
        grid=(B,),
        in_specs=[
            pl.BlockSpec((1, NPG, 1), lambda g: (g, 0, 0)),
            pl.BlockSpec((1, NPG, 1), lambda g: (g, 0, 0)),
            pl.BlockSpec((1, 1, P), lambda g: (g, 0, 0)),
        ],
        out_specs=pl.BlockSpec((1, NPG, 1), lambda g: (g, 0, 0)),
        out_shape=jax.ShapeDtypeStruct((B, NPG, 1), jnp.float32),
    )(s3, r3, ps3)


def kernel(instruction, distribution, node_attrs, edge_attrs, node_indices,
           sparse_coo_indices, edge_batch_indices, edge_indices, prop_embeds,
           Ws_property, W_state, W_relation):
    src_pad = jnp.pad(edge_indices[0], (0, GPAD))
    dsrc = _sc_gather_call(distribution, src_pad)

    eg3 = edge_batch_indices.reshape(NEBLK, EBLK, 1)
    dsrc3 = dsrc.reshape(NEBLK, EBLK, 1)
    ea3 = edge_attrs.reshape(NEBLK, EBLK, H)
    msgs = _edge_call(eg3, dsrc3, ea3, instruction, Ws_property[P - 1].T)

    dst2 = edge_indices[1].reshape(NW * NCH, CHUNK)
    zeros_np = jnp.zeros((NPG, H), jnp.float32)
    aggp = _sc_scatter_call(msgs.reshape(E, H), dst2, zeros_np)

    na4 = node_attrs.reshape(B, NPG, P - 1, H)
    ns3, ps3 = _node_call(na4, instruction, prop_embeds,
                          jnp.swapaxes(Ws_property[:P - 1], 1, 2))
    prop_similarities = ps3.reshape(B, P)

    wst_big = jnp.concatenate([W_state[:, None], jnp.zeros((H, H - 1), jnp.float32)], axis=1)
    wrel_big = jnp.concatenate([W_relation[:, None], jnp.zeros((H, H - 1), jnp.float32)], axis=1)
    s3, r3 = _matvec_call(ns3.reshape(NMVB, MVB, H),
                          aggp.reshape(2, NMVB, MVB, H),
                          wst_big, wrel_big)

    out = _finish_call(s3.reshape(B, NPG, 1), r3.reshape(B, NPG, 1), ps3)
    return (out.reshape(N), prop_similarities)
